# baseline (speedup 1.0000x reference)
import functools

import jax
import jax.numpy as jnp
from jax import lax
from jax.experimental import pallas as pl
from jax.experimental.pallas import tpu as pltpu
from jax.experimental.pallas import tpu_sc as plsc

N_NODES = 100000
N_EDGES = 6400000
FLOW_ITERS = 10

L = 16
NC = 2
NS = 16
NW = NC * NS

CHUNK = 1024
IDXW = 128
RPC = CHUNK // IDXW
NCHUNK_W = 198
NBUF = 3
E_PAD = NW * NCHUNK_W * CHUNK
NP = 102400
SLICE = NP // NS
ZW = 1600
WIN = 5120

_MESH = plsc.VectorSubcoreMesh(core_axis_name="c", subcore_axis_name="s",
                               num_cores=NC, num_subcores=NS)
_SC_PARAMS = pltpu.CompilerParams(needs_layout_passes=False)

_GDN = lax.GatherDimensionNumbers(offset_dims=(), collapsed_slice_dims=(0,),
                                  start_index_map=(0,))


def _vgather(x, idx):
    return lax.gather(x, idx[:, None], _GDN, (1,),
                      mode=lax.GatherScatterMode.PROMISE_IN_BOUNDS)


def _compute_chunk(adj_l, rows_st, pred_st, con_st):
    @plsc.parallel_loop(0, CHUNK // L, unroll=8)
    def _vec(i):
        rvec = rows_st[pl.ds(i * L, L)]
        g = plsc.load_gather(adj_l, [rvec])
        con_st[i // (IDXW // L), pl.ds((i % (IDXW // L)) * L, L)] = (
            g * pred_st[pl.ds(i * L, L)])


@functools.partial(
    pl.kernel,
    out_type=jax.ShapeDtypeStruct((NC, NP), jnp.float32),
    mesh=_MESH,
    compiler_params=_SC_PARAMS,
    scratch_types=[
        pltpu.VMEM((NP,), jnp.float32),
        [pltpu.VMEM((CHUNK,), jnp.int32) for _ in range(NBUF)],
        [pltpu.VMEM((CHUNK,), jnp.float32) for _ in range(NBUF)],
        [pltpu.VMEM((CHUNK,), jnp.int32) for _ in range(NBUF)],
        pltpu.VMEM((WIN,), jnp.float32),
        pltpu.VMEM((WIN // IDXW, IDXW), jnp.int32),
        pltpu.VMEM((L,), jnp.float32),
        pltpu.VMEM((L,), jnp.float32),
        pltpu.VMEM((ZW,), jnp.float32),
        pltpu.VMEM_SHARED((NP,), jnp.float32),
        [pltpu.SemaphoreType.DMA for _ in range(NBUF)],
        pltpu.SemaphoreType.DMA,
    ],
)
def _sc_iter(pred_hbm, rows_hbm, cols_hbm, adj_hbm, partial_hbm,
             adj_l, rows_b, pred_b, cols_b, acc, fidx, stA, stB, zeros_st,
             inflow_sh, isem, fsem):
    c = lax.axis_index("c")
    s = lax.axis_index("s")
    wid = s * NC + c
    base = wid * NCHUNK_W
    lane = jnp.arange(L, dtype=jnp.int32)
    shift_idx = jnp.minimum(lane + 1, L - 1)
    zero_idx = jnp.zeros((L,), jnp.int32)

    def _zb(i, _):
        zeros_st[pl.ds(i * L, L)] = jnp.zeros((L,), jnp.float32)
        return 0
    lax.fori_loop(0, ZW // L, _zb, 0)

    def _za(i, _):
        acc[pl.ds(i * L, L)] = jnp.zeros((L,), jnp.float32)
        return 0
    lax.fori_loop(0, WIN // L, _za, 0)

    pltpu.sync_copy(adj_hbm, adj_l)
    for z in range(SLICE // ZW):
        pltpu.sync_copy(zeros_st, inflow_sh.at[pl.ds(s * SLICE + z * ZW, ZW)])
    plsc.subcore_barrier()

    def fire_inputs(k, b):
        e0 = (base + k) * CHUNK
        pltpu.async_copy(rows_hbm.at[pl.ds(e0, CHUNK)], rows_b[b], isem[b])
        pltpu.async_copy(pred_hbm.at[pl.ds(e0, CHUNK)], pred_b[b], isem[b])
        pltpu.async_copy(cols_hbm.at[pl.ds(e0, CHUNK)], cols_b[b], isem[b])

    def wait_inputs(k, b):
        e0 = (base + k) * CHUNK
        pltpu.make_async_copy(rows_hbm.at[pl.ds(e0, CHUNK)], rows_b[b],
                              isem[b]).wait()
        pltpu.make_async_copy(pred_hbm.at[pl.ds(e0, CHUNK)], pred_b[b],
                              isem[b]).wait()
        pltpu.make_async_copy(cols_hbm.at[pl.ds(e0, CHUNK)], cols_b[b],
                              isem[b]).wait()

    fire_inputs(0, 0)

    def group(g, base_vec):
        for b in range(NBUF):
            k = g * NBUF + b
            wait_inputs(k, b)

            @pl.when(k + 1 < NCHUNK_W)
            def _():
                fire_inputs(k + 1, (b + 1) % NBUF)

            cvec0 = cols_b[b][pl.ds(0, L)]
            base_vec = jnp.where(k == 0, _vgather(cvec0, zero_idx), base_vec)

            @plsc.parallel_loop(0, CHUNK // L, unroll=4)
            def _vec(i):
                rvec = rows_b[b][pl.ds(i * L, L)]
                gth = plsc.load_gather(adj_l, [rvec])
                contrib = gth * pred_b[b][pl.ds(i * L, L)]
                cvec = cols_b[b][pl.ds(i * L, L)]
                offs_a = cvec - base_vec
                plsc.addupdate_scatter(acc, [offs_a], contrib,
                                       mask=offs_a < WIN)

            tail = cols_b[b][pl.ds(CHUNK - L, L)] - base_vec
            ov = lax.reduce_max(tail, (0,)) >= WIN

            @pl.when(ov)
            def _():
                def slow(i, _):
                    rvec = rows_b[b][pl.ds(i * L, L)]
                    gth = plsc.load_gather(adj_l, [rvec])
                    contrib = gth * pred_b[b][pl.ds(i * L, L)]
                    cvec = cols_b[b][pl.ds(i * L, L)]
                    out_a = (cvec - base_vec) >= WIN
                    stA[pl.ds(0, L)] = jnp.where(out_a, contrib, 0.0)
                    pltpu.sync_copy(stA, inflow_sh.at[cvec], add=True)
                    return 0
                lax.fori_loop(0, CHUNK // L, slow, 0)
        return base_vec
    base_vec = lax.fori_loop(0, NCHUNK_W // NBUF, group,
                             jnp.zeros((L,), jnp.int32))

    def _fill(i, _):
        vals = jnp.minimum(base_vec + i * L + lane, NP - 1)
        fidx[i // (IDXW // L), pl.ds((i % (IDXW // L)) * L, L)] = vals
        return 0
    lax.fori_loop(0, WIN // L, _fill, 0)
    for r in range(WIN // IDXW):
        pltpu.async_copy(acc.at[pl.ds(r * IDXW, IDXW)],
                         inflow_sh.at[fidx.at[r]], fsem, add=True)
    for r in range(WIN // IDXW):
        pltpu.make_async_copy(acc.at[pl.ds(r * IDXW, IDXW)],
                              inflow_sh.at[fidx.at[r]], fsem).wait()

    plsc.subcore_barrier()
    pltpu.sync_copy(inflow_sh.at[pl.ds(s * SLICE, SLICE)],
                    partial_hbm.at[c, pl.ds(s * SLICE, SLICE)])


@functools.partial(
    pl.kernel,
    out_type=jax.ShapeDtypeStruct((E_PAD // IDXW, IDXW), jnp.float32),
    mesh=_MESH,
    compiler_params=_SC_PARAMS,
    scratch_types=[
        pltpu.VMEM((NP,), jnp.float32),
        [pltpu.VMEM((CHUNK,), jnp.int32) for _ in range(NBUF)],
        [pltpu.VMEM((CHUNK,), jnp.float32) for _ in range(NBUF)],
        [pltpu.VMEM((RPC, IDXW), jnp.float32) for _ in range(NBUF)],
        [pltpu.SemaphoreType.DMA for _ in range(NBUF)],
        [pltpu.SemaphoreType.DMA for _ in range(NBUF)],
    ],
)
def _sc_final(pred_hbm, rows_hbm, adj_hbm, out_hbm,
              adj_l, rows_b, pred_b, con_b, isem, osem):
    c = lax.axis_index("c")
    s = lax.axis_index("s")
    wid = s * NC + c
    base = wid * NCHUNK_W
    pltpu.sync_copy(adj_hbm, adj_l)

    def fire_inputs(k, b):
        e0 = (base + k) * CHUNK
        pltpu.async_copy(rows_hbm.at[pl.ds(e0, CHUNK)], rows_b[b], isem[b])
        pltpu.async_copy(pred_hbm.at[pl.ds(e0, CHUNK)], pred_b[b], isem[b])

    def wait_inputs(k, b):
        e0 = (base + k) * CHUNK
        pltpu.make_async_copy(rows_hbm.at[pl.ds(e0, CHUNK)], rows_b[b],
                              isem[b]).wait()
        pltpu.make_async_copy(pred_hbm.at[pl.ds(e0, CHUNK)], pred_b[b],
                              isem[b]).wait()

    def drain_out(k, b):
        r0 = (base + k) * RPC
        pltpu.make_async_copy(con_b[b], out_hbm.at[pl.ds(r0, RPC)],
                              osem[b]).wait()

    fire_inputs(0, 0)

    def group(g, _):
        for b in range(NBUF):
            k = g * NBUF + b

            wait_inputs(k, b)

            @pl.when(k >= 2)
            def _():
                drain_out(k - 2, (b + 1) % NBUF)

            @pl.when(k + 1 < NCHUNK_W)
            def _():
                fire_inputs(k + 1, (b + 1) % NBUF)

            _compute_chunk(adj_l, rows_b[b], pred_b[b], con_b[b])
            r0 = (base + k) * RPC
            pltpu.async_copy(con_b[b], out_hbm.at[pl.ds(r0, RPC)], osem[b])
        return 0
    lax.fori_loop(0, NCHUNK_W // NBUF, group, 0)

    drain_out(NCHUNK_W - 2, (NCHUNK_W - 2) % NBUF)
    drain_out(NCHUNK_W - 1, (NCHUNK_W - 1) % NBUF)


def _combine_body(p_ref, d_ref, o_ref):
    o_ref[...] = jnp.maximum(p_ref[0] + p_ref[1] - d_ref[...], 0.0)


def _tc_combine(partial, demands_pad):
    out = pl.pallas_call(
        _combine_body,
        out_shape=jax.ShapeDtypeStruct((NP // 128, 128), jnp.float32),
    )(partial.reshape(NC, NP // 128, 128),
      demands_pad.reshape(NP // 128, 128))
    return out.reshape(NP)


def kernel(values, rows, cols, demands):
    pred = jnp.pad(values, (0, E_PAD - N_EDGES))
    rows_p = jnp.pad(rows, (0, E_PAD - N_EDGES))
    cols_p = jnp.pad(cols, (0, E_PAD - N_EDGES), constant_values=N_NODES)
    d_pad = jnp.pad(demands[:, 0], (0, NP - N_NODES))

    adj = jnp.ones((NP,), jnp.float32)
    for _ in range(FLOW_ITERS):
        partial = _sc_iter(pred, rows_p, cols_p, adj)
        adj = _tc_combine(partial, d_pad)
    out = _sc_final(pred, rows_p, adj)
    return out.reshape(E_PAD)[:N_EDGES]


# speedup vs baseline: 1.5435x; 1.5435x over previous
import functools

import jax
import jax.numpy as jnp
from jax import lax
from jax.experimental import pallas as pl
from jax.experimental.pallas import tpu as pltpu
from jax.experimental.pallas import tpu_sc as plsc

N_NODES = 100000
N_EDGES = 6400000
FLOW_ITERS = 10

L = 16
NC = 2
NS = 16
NW = NC * NS

CHUNK = 1024
IDXW = 128
RPC = CHUNK // IDXW
NCHUNK_W = 198
NBUF = 3
E_PAD = NW * NCHUNK_W * CHUNK
NP = 102400
SLICE = NP // NS
ZW = 1600
WIN = 5120

_MESH = plsc.VectorSubcoreMesh(core_axis_name="c", subcore_axis_name="s",
                               num_cores=NC, num_subcores=NS)
_SC_PARAMS = pltpu.CompilerParams(needs_layout_passes=False)

_GDN = lax.GatherDimensionNumbers(offset_dims=(), collapsed_slice_dims=(0,),
                                  start_index_map=(0,))


def _vgather(x, idx):
    return lax.gather(x, idx[:, None], _GDN, (1,),
                      mode=lax.GatherScatterMode.PROMISE_IN_BOUNDS)


def _compute_chunk(adj_l, rows_st, pred_st, con_st):
    @plsc.parallel_loop(0, CHUNK // L, unroll=8)
    def _vec(i):
        rvec = rows_st[pl.ds(i * L, L)]
        g = plsc.load_gather(adj_l, [rvec])
        con_st[i // (IDXW // L), pl.ds((i % (IDXW // L)) * L, L)] = (
            g * pred_st[pl.ds(i * L, L)])


@functools.partial(
    pl.kernel,
    out_type=jax.ShapeDtypeStruct((NC, NP), jnp.float32),
    mesh=_MESH,
    compiler_params=_SC_PARAMS,
    scratch_types=[
        pltpu.VMEM((NP,), jnp.float32),
        [pltpu.VMEM((CHUNK,), jnp.int32) for _ in range(NBUF)],
        [pltpu.VMEM((CHUNK,), jnp.float32) for _ in range(NBUF)],
        [pltpu.VMEM((CHUNK,), jnp.int32) for _ in range(NBUF)],
        pltpu.VMEM((WIN,), jnp.float32),
        pltpu.VMEM((WIN // IDXW, IDXW), jnp.int32),
        pltpu.VMEM((L,), jnp.float32),
        pltpu.VMEM((L,), jnp.float32),
        pltpu.VMEM((ZW,), jnp.float32),
        pltpu.VMEM_SHARED((NP,), jnp.float32),
        [pltpu.SemaphoreType.DMA for _ in range(NBUF)],
        pltpu.SemaphoreType.DMA,
    ],
)
def _sc_iter(pred_hbm, rows_hbm, cols_hbm, adj_hbm, partial_hbm,
             adj_l, rows_b, pred_b, cols_b, acc, fidx, stA, stB, zeros_st,
             inflow_sh, isem, fsem):
    c = lax.axis_index("c")
    s = lax.axis_index("s")
    wid = s * NC + c
    base = wid * NCHUNK_W
    lane = jnp.arange(L, dtype=jnp.int32)
    shift_idx = jnp.minimum(lane + 1, L - 1)
    zero_idx = jnp.zeros((L,), jnp.int32)

    def _zb(i, _):
        zeros_st[pl.ds(i * L, L)] = jnp.zeros((L,), jnp.float32)
        return 0
    lax.fori_loop(0, ZW // L, _zb, 0)

    def _za(i, _):
        acc[pl.ds(i * L, L)] = jnp.zeros((L,), jnp.float32)
        return 0
    lax.fori_loop(0, WIN // L, _za, 0)

    pltpu.sync_copy(adj_hbm, adj_l)
    for z in range(SLICE // ZW):
        pltpu.sync_copy(zeros_st, inflow_sh.at[pl.ds(s * SLICE + z * ZW, ZW)])
    plsc.subcore_barrier()

    def fire_inputs(k, b):
        e0 = (base + k) * CHUNK
        pltpu.async_copy(rows_hbm.at[pl.ds(e0, CHUNK)], rows_b[b], isem[b])
        pltpu.async_copy(pred_hbm.at[pl.ds(e0, CHUNK)], pred_b[b], isem[b])
        pltpu.async_copy(cols_hbm.at[pl.ds(e0, CHUNK)], cols_b[b], isem[b])

    def wait_inputs(k, b):
        e0 = (base + k) * CHUNK
        pltpu.make_async_copy(rows_hbm.at[pl.ds(e0, CHUNK)], rows_b[b],
                              isem[b]).wait()
        pltpu.make_async_copy(pred_hbm.at[pl.ds(e0, CHUNK)], pred_b[b],
                              isem[b]).wait()
        pltpu.make_async_copy(cols_hbm.at[pl.ds(e0, CHUNK)], cols_b[b],
                              isem[b]).wait()

    fire_inputs(0, 0)

    def group(g, base_vec):
        for b in range(NBUF):
            k = g * NBUF + b
            wait_inputs(k, b)

            @pl.when(k + 1 < NCHUNK_W)
            def _():
                fire_inputs(k + 1, (b + 1) % NBUF)

            cvec0 = cols_b[b][pl.ds(0, L)]
            base_vec = jnp.where(k == 0, _vgather(cvec0, zero_idx), base_vec)

            @plsc.parallel_loop(0, CHUNK // L, unroll=4)
            def _vec(i):
                rvec = rows_b[b][pl.ds(i * L, L)]
                gth = plsc.load_gather(adj_l, [rvec])
                contrib = gth * pred_b[b][pl.ds(i * L, L)]
                cvec = cols_b[b][pl.ds(i * L, L)]
                ps = contrib
                for d in (1, 2, 4, 8):
                    sh = _vgather(ps, jnp.maximum(lane - d, 0))
                    ps = ps + jnp.where(lane >= d, sh, 0.0)
                cnext = _vgather(cvec, shift_idx)
                run_end = cvec != cnext
                mask_a = run_end | (lane == L - 1)
                offs_a = cvec - base_vec
                plsc.addupdate_scatter(acc, [offs_a], ps,
                                       mask=mask_a & (offs_a < WIN))
                offs_b = cnext - base_vec
                plsc.addupdate_scatter(acc, [offs_b], 0.0 - ps,
                                       mask=run_end & (offs_b < WIN))

            tail = cols_b[b][pl.ds(CHUNK - L, L)] - base_vec
            ov = lax.reduce_max(tail, (0,)) >= WIN

            @pl.when(ov)
            def _():
                def slow(i, _):
                    rvec = rows_b[b][pl.ds(i * L, L)]
                    gth = plsc.load_gather(adj_l, [rvec])
                    contrib = gth * pred_b[b][pl.ds(i * L, L)]
                    cvec = cols_b[b][pl.ds(i * L, L)]
                    ps = contrib
                    for d in (1, 2, 4, 8):
                        sh = _vgather(ps, jnp.maximum(lane - d, 0))
                        ps = ps + jnp.where(lane >= d, sh, 0.0)
                    cnext = _vgather(cvec, shift_idx)
                    run_end = cvec != cnext
                    mask_a = run_end | (lane == L - 1)
                    out_a = mask_a & ((cvec - base_vec) >= WIN)
                    out_b = run_end & ((cnext - base_vec) >= WIN)
                    stA[pl.ds(0, L)] = jnp.where(out_a, ps, 0.0)
                    pltpu.sync_copy(stA, inflow_sh.at[cvec], add=True)
                    stB[pl.ds(0, L)] = jnp.where(out_b, 0.0 - ps, 0.0)
                    pltpu.sync_copy(stB, inflow_sh.at[cnext], add=True)
                    return 0
                lax.fori_loop(0, CHUNK // L, slow, 0)
        return base_vec
    base_vec = lax.fori_loop(0, NCHUNK_W // NBUF, group,
                             jnp.zeros((L,), jnp.int32))

    def _fill(i, _):
        vals = jnp.minimum(base_vec + i * L + lane, NP - 1)
        fidx[i // (IDXW // L), pl.ds((i % (IDXW // L)) * L, L)] = vals
        return 0
    lax.fori_loop(0, WIN // L, _fill, 0)
    for r in range(WIN // IDXW):
        pltpu.async_copy(acc.at[pl.ds(r * IDXW, IDXW)],
                         inflow_sh.at[fidx.at[r]], fsem, add=True)
    for r in range(WIN // IDXW):
        pltpu.make_async_copy(acc.at[pl.ds(r * IDXW, IDXW)],
                              inflow_sh.at[fidx.at[r]], fsem).wait()

    plsc.subcore_barrier()
    pltpu.sync_copy(inflow_sh.at[pl.ds(s * SLICE, SLICE)],
                    partial_hbm.at[c, pl.ds(s * SLICE, SLICE)])


@functools.partial(
    pl.kernel,
    out_type=jax.ShapeDtypeStruct((E_PAD // IDXW, IDXW), jnp.float32),
    mesh=_MESH,
    compiler_params=_SC_PARAMS,
    scratch_types=[
        pltpu.VMEM((NP,), jnp.float32),
        [pltpu.VMEM((CHUNK,), jnp.int32) for _ in range(NBUF)],
        [pltpu.VMEM((CHUNK,), jnp.float32) for _ in range(NBUF)],
        [pltpu.VMEM((RPC, IDXW), jnp.float32) for _ in range(NBUF)],
        [pltpu.SemaphoreType.DMA for _ in range(NBUF)],
        [pltpu.SemaphoreType.DMA for _ in range(NBUF)],
    ],
)
def _sc_final(pred_hbm, rows_hbm, adj_hbm, out_hbm,
              adj_l, rows_b, pred_b, con_b, isem, osem):
    c = lax.axis_index("c")
    s = lax.axis_index("s")
    wid = s * NC + c
    base = wid * NCHUNK_W
    pltpu.sync_copy(adj_hbm, adj_l)

    def fire_inputs(k, b):
        e0 = (base + k) * CHUNK
        pltpu.async_copy(rows_hbm.at[pl.ds(e0, CHUNK)], rows_b[b], isem[b])
        pltpu.async_copy(pred_hbm.at[pl.ds(e0, CHUNK)], pred_b[b], isem[b])

    def wait_inputs(k, b):
        e0 = (base + k) * CHUNK
        pltpu.make_async_copy(rows_hbm.at[pl.ds(e0, CHUNK)], rows_b[b],
                              isem[b]).wait()
        pltpu.make_async_copy(pred_hbm.at[pl.ds(e0, CHUNK)], pred_b[b],
                              isem[b]).wait()

    def drain_out(k, b):
        r0 = (base + k) * RPC
        pltpu.make_async_copy(con_b[b], out_hbm.at[pl.ds(r0, RPC)],
                              osem[b]).wait()

    fire_inputs(0, 0)

    def group(g, _):
        for b in range(NBUF):
            k = g * NBUF + b

            wait_inputs(k, b)

            @pl.when(k >= 2)
            def _():
                drain_out(k - 2, (b + 1) % NBUF)

            @pl.when(k + 1 < NCHUNK_W)
            def _():
                fire_inputs(k + 1, (b + 1) % NBUF)

            _compute_chunk(adj_l, rows_b[b], pred_b[b], con_b[b])
            r0 = (base + k) * RPC
            pltpu.async_copy(con_b[b], out_hbm.at[pl.ds(r0, RPC)], osem[b])
        return 0
    lax.fori_loop(0, NCHUNK_W // NBUF, group, 0)

    drain_out(NCHUNK_W - 2, (NCHUNK_W - 2) % NBUF)
    drain_out(NCHUNK_W - 1, (NCHUNK_W - 1) % NBUF)


def _combine_body(p_ref, d_ref, o_ref):
    o_ref[...] = jnp.maximum(p_ref[0] + p_ref[1] - d_ref[...], 0.0)


def _tc_combine(partial, demands_pad):
    out = pl.pallas_call(
        _combine_body,
        out_shape=jax.ShapeDtypeStruct((NP // 128, 128), jnp.float32),
    )(partial.reshape(NC, NP // 128, 128),
      demands_pad.reshape(NP // 128, 128))
    return out.reshape(NP)


def kernel(values, rows, cols, demands):
    pred = jnp.pad(values, (0, E_PAD - N_EDGES))
    rows_p = jnp.pad(rows, (0, E_PAD - N_EDGES))
    cols_p = jnp.pad(cols, (0, E_PAD - N_EDGES), constant_values=N_NODES)
    d_pad = jnp.pad(demands[:, 0], (0, NP - N_NODES))

    adj = jnp.ones((NP,), jnp.float32)
    for _ in range(FLOW_ITERS):
        partial = _sc_iter(pred, rows_p, cols_p, adj)
        adj = _tc_combine(partial, d_pad)
    out = _sc_final(pred, rows_p, adj)
    return out.reshape(E_PAD)[:N_EDGES]


# speedup vs baseline: 1.8187x; 1.1783x over previous
import functools

import jax
import jax.numpy as jnp
from jax import lax
from jax.experimental import pallas as pl
from jax.experimental.pallas import tpu as pltpu
from jax.experimental.pallas import tpu_sc as plsc

N_NODES = 100000
N_EDGES = 6400000
FLOW_ITERS = 10

L = 16
NC = 2
NS = 16
NW = NC * NS

CHUNK = 1024
IDXW = 128
RPC = CHUNK // IDXW
NCHUNK_W = 198
NBUF = 3
E_PAD = NW * NCHUNK_W * CHUNK
NP = 102400
SLICE = NP // NS
ZW = 1600
WIN = 5120

_MESH = plsc.VectorSubcoreMesh(core_axis_name="c", subcore_axis_name="s",
                               num_cores=NC, num_subcores=NS)
_SC_PARAMS = pltpu.CompilerParams(needs_layout_passes=False)

_GDN = lax.GatherDimensionNumbers(offset_dims=(), collapsed_slice_dims=(0,),
                                  start_index_map=(0,))


def _vgather(x, idx):
    return lax.gather(x, idx[:, None], _GDN, (1,),
                      mode=lax.GatherScatterMode.PROMISE_IN_BOUNDS)


def _compute_chunk(adj_l, rows_st, pred_st, con_st):
    @plsc.parallel_loop(0, CHUNK // L, unroll=8)
    def _vec(i):
        rvec = rows_st[pl.ds(i * L, L)]
        g = plsc.load_gather(adj_l, [rvec])
        con_st[i // (IDXW // L), pl.ds((i % (IDXW // L)) * L, L)] = (
            g * pred_st[pl.ds(i * L, L)])


@functools.partial(
    pl.kernel,
    out_type=jax.ShapeDtypeStruct((NC, NP), jnp.float32),
    mesh=_MESH,
    compiler_params=_SC_PARAMS,
    scratch_types=[
        pltpu.VMEM((NP,), jnp.float32),
        [pltpu.VMEM((CHUNK,), jnp.int32) for _ in range(NBUF)],
        [pltpu.VMEM((CHUNK,), jnp.float32) for _ in range(NBUF)],
        [pltpu.VMEM((CHUNK,), jnp.int32) for _ in range(NBUF)],
        pltpu.VMEM((WIN,), jnp.float32),
        pltpu.VMEM((WIN // IDXW, IDXW), jnp.int32),
        pltpu.VMEM((L,), jnp.float32),
        pltpu.VMEM((L,), jnp.float32),
        pltpu.VMEM((ZW,), jnp.float32),
        pltpu.VMEM_SHARED((NP,), jnp.float32),
        [pltpu.SemaphoreType.DMA for _ in range(NBUF)],
        pltpu.SemaphoreType.DMA,
    ],
)
def _sc_iter(pred_hbm, rows_hbm, cols_hbm, adj_hbm, partial_hbm,
             adj_l, rows_b, pred_b, cols_b, acc, fidx, stA, stB, zeros_st,
             inflow_sh, isem, fsem):
    c = lax.axis_index("c")
    s = lax.axis_index("s")
    wid = s * NC + c
    base = wid * NCHUNK_W
    lane = jnp.arange(L, dtype=jnp.int32)
    shift_idx = jnp.minimum(lane + 1, L - 1)
    zero_idx = jnp.zeros((L,), jnp.int32)

    def _zb(i, _):
        zeros_st[pl.ds(i * L, L)] = jnp.zeros((L,), jnp.float32)
        return 0
    lax.fori_loop(0, ZW // L, _zb, 0)

    def _za(i, _):
        acc[pl.ds(i * L, L)] = jnp.zeros((L,), jnp.float32)
        return 0
    lax.fori_loop(0, WIN // L, _za, 0)

    pltpu.sync_copy(adj_hbm, adj_l)
    for z in range(SLICE // ZW):
        pltpu.sync_copy(zeros_st, inflow_sh.at[pl.ds(s * SLICE + z * ZW, ZW)])
    plsc.subcore_barrier()

    def fire_inputs(k, b):
        e0 = (base + k) * CHUNK
        pltpu.async_copy(rows_hbm.at[pl.ds(e0, CHUNK)], rows_b[b], isem[b])
        pltpu.async_copy(pred_hbm.at[pl.ds(e0, CHUNK)], pred_b[b], isem[b])
        pltpu.async_copy(cols_hbm.at[pl.ds(e0, CHUNK)], cols_b[b], isem[b])

    def wait_inputs(k, b):
        e0 = (base + k) * CHUNK
        pltpu.make_async_copy(rows_hbm.at[pl.ds(e0, CHUNK)], rows_b[b],
                              isem[b]).wait()
        pltpu.make_async_copy(pred_hbm.at[pl.ds(e0, CHUNK)], pred_b[b],
                              isem[b]).wait()
        pltpu.make_async_copy(cols_hbm.at[pl.ds(e0, CHUNK)], cols_b[b],
                              isem[b]).wait()

    fire_inputs(0, 0)
    fire_inputs(1, 1)

    def group(g, base_vec):
        for b in range(NBUF):
            k = g * NBUF + b
            wait_inputs(k, b)

            @pl.when(k + 2 < NCHUNK_W)
            def _():
                fire_inputs(k + 2, (b + 2) % NBUF)

            cvec0 = cols_b[b][pl.ds(0, L)]
            base_vec = jnp.where(k == 0, _vgather(cvec0, zero_idx), base_vec)

            @plsc.parallel_loop(0, CHUNK // L, unroll=4)
            def _vec(i):
                rvec = rows_b[b][pl.ds(i * L, L)]
                gth = plsc.load_gather(adj_l, [rvec])
                contrib = gth * pred_b[b][pl.ds(i * L, L)]
                cvec = cols_b[b][pl.ds(i * L, L)]
                ps = contrib
                for d in (1, 2, 4, 8):
                    sh = _vgather(ps, jnp.maximum(lane - d, 0))
                    ps = ps + jnp.where(lane >= d, sh, 0.0)
                cnext = _vgather(cvec, shift_idx)
                run_end = cvec != cnext
                mask_a = run_end | (lane == L - 1)
                offs_a = cvec - base_vec
                plsc.addupdate_scatter(acc, [offs_a], ps,
                                       mask=mask_a & (offs_a < WIN))
                offs_b = cnext - base_vec
                plsc.addupdate_scatter(acc, [offs_b], 0.0 - ps,
                                       mask=run_end & (offs_b < WIN))

            tail = cols_b[b][pl.ds(CHUNK - L, L)] - base_vec
            ov = lax.reduce_max(tail, (0,)) >= WIN

            @pl.when(ov)
            def _():
                def slow(i, _):
                    rvec = rows_b[b][pl.ds(i * L, L)]
                    gth = plsc.load_gather(adj_l, [rvec])
                    contrib = gth * pred_b[b][pl.ds(i * L, L)]
                    cvec = cols_b[b][pl.ds(i * L, L)]
                    ps = contrib
                    for d in (1, 2, 4, 8):
                        sh = _vgather(ps, jnp.maximum(lane - d, 0))
                        ps = ps + jnp.where(lane >= d, sh, 0.0)
                    cnext = _vgather(cvec, shift_idx)
                    run_end = cvec != cnext
                    mask_a = run_end | (lane == L - 1)
                    out_a = mask_a & ((cvec - base_vec) >= WIN)
                    out_b = run_end & ((cnext - base_vec) >= WIN)
                    stA[pl.ds(0, L)] = jnp.where(out_a, ps, 0.0)
                    pltpu.sync_copy(stA, inflow_sh.at[cvec], add=True)
                    stB[pl.ds(0, L)] = jnp.where(out_b, 0.0 - ps, 0.0)
                    pltpu.sync_copy(stB, inflow_sh.at[cnext], add=True)
                    return 0
                lax.fori_loop(0, CHUNK // L, slow, 0)
        return base_vec
    base_vec = lax.fori_loop(0, NCHUNK_W // NBUF, group,
                             jnp.zeros((L,), jnp.int32))

    def _fill(i, _):
        vals = jnp.minimum(base_vec + i * L + lane, NP - 1)
        fidx[i // (IDXW // L), pl.ds((i % (IDXW // L)) * L, L)] = vals
        return 0
    lax.fori_loop(0, WIN // L, _fill, 0)
    for r in range(WIN // IDXW):
        pltpu.async_copy(acc.at[pl.ds(r * IDXW, IDXW)],
                         inflow_sh.at[fidx.at[r]], fsem, add=True)
    for r in range(WIN // IDXW):
        pltpu.make_async_copy(acc.at[pl.ds(r * IDXW, IDXW)],
                              inflow_sh.at[fidx.at[r]], fsem).wait()

    plsc.subcore_barrier()
    pltpu.sync_copy(inflow_sh.at[pl.ds(s * SLICE, SLICE)],
                    partial_hbm.at[c, pl.ds(s * SLICE, SLICE)])


@functools.partial(
    pl.kernel,
    out_type=jax.ShapeDtypeStruct((E_PAD // IDXW, IDXW), jnp.float32),
    mesh=_MESH,
    compiler_params=_SC_PARAMS,
    scratch_types=[
        pltpu.VMEM((NP,), jnp.float32),
        [pltpu.VMEM((CHUNK,), jnp.int32) for _ in range(NBUF)],
        [pltpu.VMEM((CHUNK,), jnp.float32) for _ in range(NBUF)],
        [pltpu.VMEM((RPC, IDXW), jnp.float32) for _ in range(NBUF)],
        [pltpu.SemaphoreType.DMA for _ in range(NBUF)],
        [pltpu.SemaphoreType.DMA for _ in range(NBUF)],
    ],
)
def _sc_final(pred_hbm, rows_hbm, adj_hbm, out_hbm,
              adj_l, rows_b, pred_b, con_b, isem, osem):
    c = lax.axis_index("c")
    s = lax.axis_index("s")
    wid = s * NC + c
    base = wid * NCHUNK_W
    pltpu.sync_copy(adj_hbm, adj_l)

    def fire_inputs(k, b):
        e0 = (base + k) * CHUNK
        pltpu.async_copy(rows_hbm.at[pl.ds(e0, CHUNK)], rows_b[b], isem[b])
        pltpu.async_copy(pred_hbm.at[pl.ds(e0, CHUNK)], pred_b[b], isem[b])

    def wait_inputs(k, b):
        e0 = (base + k) * CHUNK
        pltpu.make_async_copy(rows_hbm.at[pl.ds(e0, CHUNK)], rows_b[b],
                              isem[b]).wait()
        pltpu.make_async_copy(pred_hbm.at[pl.ds(e0, CHUNK)], pred_b[b],
                              isem[b]).wait()

    def drain_out(k, b):
        r0 = (base + k) * RPC
        pltpu.make_async_copy(con_b[b], out_hbm.at[pl.ds(r0, RPC)],
                              osem[b]).wait()

    fire_inputs(0, 0)

    def group(g, _):
        for b in range(NBUF):
            k = g * NBUF + b

            wait_inputs(k, b)

            @pl.when(k >= 2)
            def _():
                drain_out(k - 2, (b + 1) % NBUF)

            @pl.when(k + 1 < NCHUNK_W)
            def _():
                fire_inputs(k + 1, (b + 1) % NBUF)

            _compute_chunk(adj_l, rows_b[b], pred_b[b], con_b[b])
            r0 = (base + k) * RPC
            pltpu.async_copy(con_b[b], out_hbm.at[pl.ds(r0, RPC)], osem[b])
        return 0
    lax.fori_loop(0, NCHUNK_W // NBUF, group, 0)

    drain_out(NCHUNK_W - 2, (NCHUNK_W - 2) % NBUF)
    drain_out(NCHUNK_W - 1, (NCHUNK_W - 1) % NBUF)


def _combine_body(p_ref, d_ref, o_ref):
    o_ref[...] = jnp.maximum(p_ref[0] + p_ref[1] - d_ref[...], 0.0)


def _tc_combine(partial, demands_pad):
    out = pl.pallas_call(
        _combine_body,
        out_shape=jax.ShapeDtypeStruct((NP // 128, 128), jnp.float32),
    )(partial.reshape(NC, NP // 128, 128),
      demands_pad.reshape(NP // 128, 128))
    return out.reshape(NP)


def kernel(values, rows, cols, demands):
    pred = jnp.pad(values, (0, E_PAD - N_EDGES))
    rows_p = jnp.pad(rows, (0, E_PAD - N_EDGES))
    cols_p = jnp.pad(cols, (0, E_PAD - N_EDGES), constant_values=N_NODES)
    d_pad = jnp.pad(demands[:, 0], (0, NP - N_NODES))

    adj = jnp.ones((NP,), jnp.float32)
    for _ in range(FLOW_ITERS):
        partial = _sc_iter(pred, rows_p, cols_p, adj)
        adj = _tc_combine(partial, d_pad)
    out = _sc_final(pred, rows_p, adj)
    return out.reshape(E_PAD)[:N_EDGES]
